# Initial kernel scaffold; baseline (speedup 1.0000x reference)
import jax
import jax.numpy as jnp
from jax.experimental import pallas as pl


def kernel(x, adj, W1, W2, W3):
    raise NotImplementedError("write your pallas kernel here")


# speedup vs baseline: 1.1899x; 1.1899x over previous
import functools

import jax
import jax.numpy as jnp
from jax.experimental import pallas as pl
from jax.experimental.pallas import tpu as pltpu


def _pick_block(n: int, target: int) -> int:
    best = 0
    for d in range(8, min(n, target) + 1, 8):
        if n % d == 0:
            best = d
    return best if best else n


def _proj_body(a_ref, w_ref, o_ref):
    o_ref[...] = jnp.dot(a_ref[...], w_ref[...],
                         preferred_element_type=jnp.float32)


def _proj(a, w, bm):
    n, k = a.shape
    f = w.shape[1]
    return pl.pallas_call(
        _proj_body,
        grid=(n // bm,),
        in_specs=[
            pl.BlockSpec((bm, k), lambda i: (i, 0)),
            pl.BlockSpec((k, f), lambda i: (0, 0)),
        ],
        out_specs=pl.BlockSpec((bm, f), lambda i: (i, 0)),
        out_shape=jax.ShapeDtypeStruct((n, f), jnp.float32),
        compiler_params=pltpu.CompilerParams(
            dimension_semantics=("parallel",)),
    )(a, w)


def _adjmm_body(adj_ref, h_ref, o_ref, *, relu):
    acc = jnp.dot(adj_ref[...], h_ref[...],
                  preferred_element_type=jnp.float32)
    if relu:
        acc = jnp.maximum(acc, 0.0)
    o_ref[...] = acc


def _adjmm(adj, h, bm, relu):
    n = adj.shape[0]
    f = h.shape[1]
    return pl.pallas_call(
        functools.partial(_adjmm_body, relu=relu),
        grid=(n // bm,),
        in_specs=[
            pl.BlockSpec((bm, n), lambda i: (i, 0)),
            pl.BlockSpec((n, f), lambda i: (0, 0)),
        ],
        out_specs=pl.BlockSpec((bm, f), lambda i: (i, 0)),
        out_shape=jax.ShapeDtypeStruct((n, f), jnp.float32),
        compiler_params=pltpu.CompilerParams(
            dimension_semantics=("arbitrary",)),
    )(adj, h)


def _gram_body(a_ref, b_ref, o_ref):
    o_ref[...] = jax.lax.dot_general(
        a_ref[...], b_ref[...],
        (((1,), (1,)), ((), ())),
        preferred_element_type=jnp.float32)


def _gram(z, bm):
    n, f = z.shape
    return pl.pallas_call(
        _gram_body,
        grid=(n // bm,),
        in_specs=[
            pl.BlockSpec((bm, f), lambda i: (i, 0)),
            pl.BlockSpec((n, f), lambda i: (0, 0)),
        ],
        out_specs=pl.BlockSpec((bm, n), lambda i: (i, 0)),
        out_shape=jax.ShapeDtypeStruct((n, n), jnp.float32),
        compiler_params=pltpu.CompilerParams(
            dimension_semantics=("arbitrary",)),
    )(z, z)


def kernel(x, adj, W1, W2, W3):
    n = adj.shape[0]
    nhid = W1.shape[1]

    bm_proj = _pick_block(n, 2000)
    bm_adj = _pick_block(n, 400)
    bm_gram = _pick_block(n, 400)

    h0 = _proj(x, W1, bm_proj)
    hidden1 = _adjmm(adj, h0, bm_adj, relu=True)

    wc = jnp.concatenate([W2, W3], axis=1)
    h1c = _proj(hidden1, wc, bm_proj)
    muvar = _adjmm(adj, h1c, bm_adj, relu=False)

    mu = muvar[:, :nhid]
    logvar = muvar[:, nhid:]
    recon = _gram(mu, bm_gram)
    return (recon, mu, logvar)


# speedup vs baseline: 1.2113x; 1.0179x over previous
import functools

import jax
import jax.numpy as jnp
from jax.experimental import pallas as pl
from jax.experimental.pallas import tpu as pltpu


def _pick_block(n: int, target: int) -> int:
    best = 0
    for d in range(8, min(n, target) + 1, 8):
        if n % d == 0:
            best = d
    return best if best else n


def _proj_body(a_ref, w_ref, o_ref):
    o_ref[...] = jnp.dot(a_ref[...], w_ref[...],
                         preferred_element_type=jnp.float32)


def _proj(a, w, bm):
    n, k = a.shape
    f = w.shape[1]
    return pl.pallas_call(
        _proj_body,
        grid=(n // bm,),
        in_specs=[
            pl.BlockSpec((bm, k), lambda i: (i, 0)),
            pl.BlockSpec((k, f), lambda i: (0, 0)),
        ],
        out_specs=pl.BlockSpec((bm, f), lambda i: (i, 0)),
        out_shape=jax.ShapeDtypeStruct((n, f), jnp.float32),
        compiler_params=pltpu.CompilerParams(
            dimension_semantics=("parallel",)),
    )(a, w)


def _adjmm_body(adj_ref, h_ref, o_ref, *, relu):
    acc = jnp.dot(adj_ref[...], h_ref[...],
                  preferred_element_type=jnp.float32)
    if relu:
        acc = jnp.maximum(acc, 0.0)
    o_ref[...] = acc


def _adjmm_proj_body(adj_ref, h_ref, w_ref, o_ref, *, relu):
    acc = jnp.dot(adj_ref[...], h_ref[...],
                  preferred_element_type=jnp.float32)
    if relu:
        acc = jnp.maximum(acc, 0.0)
    o_ref[...] = jnp.dot(acc, w_ref[...],
                         preferred_element_type=jnp.float32)


def _adjmm(adj, h, bm, relu, w=None):
    n = adj.shape[0]
    f = h.shape[1]
    in_specs = [
        pl.BlockSpec((bm, n), lambda i: (i, 0)),
        pl.BlockSpec((n, f), lambda i: (0, 0)),
    ]
    operands = [adj, h]
    if w is None:
        body = functools.partial(_adjmm_body, relu=relu)
        fo = f
    else:
        body = functools.partial(_adjmm_proj_body, relu=relu)
        fo = w.shape[1]
        in_specs.append(pl.BlockSpec((f, fo), lambda i: (0, 0)))
        operands.append(w)
    return pl.pallas_call(
        body,
        grid=(n // bm,),
        in_specs=in_specs,
        out_specs=pl.BlockSpec((bm, fo), lambda i: (i, 0)),
        out_shape=jax.ShapeDtypeStruct((n, fo), jnp.float32),
        compiler_params=pltpu.CompilerParams(
            dimension_semantics=("arbitrary",)),
    )(*operands)


def _gram_body(a_ref, b_ref, o_ref):
    o_ref[...] = jax.lax.dot_general(
        a_ref[...], b_ref[...],
        (((1,), (1,)), ((), ())),
        preferred_element_type=jnp.float32)


def _gram(z, bm):
    n, f = z.shape
    return pl.pallas_call(
        _gram_body,
        grid=(n // bm,),
        in_specs=[
            pl.BlockSpec((bm, f), lambda i: (i, 0)),
            pl.BlockSpec((n, f), lambda i: (0, 0)),
        ],
        out_specs=pl.BlockSpec((bm, n), lambda i: (i, 0)),
        out_shape=jax.ShapeDtypeStruct((n, n), jnp.float32),
        compiler_params=pltpu.CompilerParams(
            dimension_semantics=("arbitrary",)),
    )(z, z)


def kernel(x, adj, W1, W2, W3):
    n = adj.shape[0]
    nhid = W1.shape[1]

    bm_proj = _pick_block(n, 2000)
    bm_adj = _pick_block(n, 400)
    bm_gram = _pick_block(n, 400)

    h0 = _proj(x, W1, bm_proj)
    wc = jnp.concatenate([W2, W3], axis=1)
    h1c = _adjmm(adj, h0, bm_adj, relu=True, w=wc)
    muvar = _adjmm(adj, h1c, bm_adj, relu=False)

    mu = muvar[:, :nhid]
    logvar = muvar[:, nhid:]
    recon = _gram(mu, bm_gram)
    return (recon, mu, logvar)


# speedup vs baseline: 1.2617x; 1.0416x over previous
import functools

import jax
import jax.numpy as jnp
from jax.experimental import pallas as pl
from jax.experimental.pallas import tpu as pltpu


def _pick_block(n: int, target: int) -> int:
    best = 0
    for d in range(8, min(n, target) + 1, 8):
        if n % d == 0:
            best = d
    return best if best else n


def _vgae_body(x_ref, w1_ref, wc_ref, adj_ref, muvar_ref, recon_ref,
               h0_ref, h1c_ref, mu_ref, *, p, bm, nhid):
    i = pl.program_id(0)
    row = jax.lax.rem(i, p) * bm

    @pl.when(i == 0)
    def _proj_x():
        h0_ref[...] = jnp.dot(x_ref[...], w1_ref[...],
                              preferred_element_type=jnp.float32)

    @pl.when(i < p)
    def _phase_a():
        acc = jnp.dot(adj_ref[...], h0_ref[...],
                      preferred_element_type=jnp.float32)
        acc = jnp.maximum(acc, 0.0)
        h1c_ref[pl.ds(row, bm), :] = jnp.dot(
            acc, wc_ref[...], preferred_element_type=jnp.float32)

    @pl.when((i >= p) & (i < 2 * p))
    def _phase_b():
        mv = jnp.dot(adj_ref[...], h1c_ref[...],
                     preferred_element_type=jnp.float32)
        muvar_ref[...] = mv
        mu_ref[pl.ds(row, bm), :] = mv[:, :nhid]

    @pl.when(i >= 2 * p)
    def _phase_c():
        recon_ref[...] = jax.lax.dot_general(
            mu_ref[pl.ds(row, bm), :], mu_ref[...],
            (((1,), (1,)), ((), ())),
            preferred_element_type=jnp.float32)


def kernel(x, adj, W1, W2, W3):
    n = adj.shape[0]
    nfeat = x.shape[1]
    nhid = W1.shape[1]
    bm = _pick_block(n, 200)
    p = n // bm

    wc = jnp.concatenate([W2, W3], axis=1)

    def adj_map(i):
        return (jnp.where(i < 2 * p, jax.lax.rem(i, p), p - 1), 0)

    def muvar_map(i):
        return (jnp.clip(i - p, 0, p - 1), 0)

    def recon_map(i):
        return (jnp.clip(i - 2 * p, 0, p - 1), 0)

    muvar, recon = pl.pallas_call(
        functools.partial(_vgae_body, p=p, bm=bm, nhid=nhid),
        grid=(3 * p,),
        in_specs=[
            pl.BlockSpec((n, nfeat), lambda i: (0, 0)),
            pl.BlockSpec((nfeat, nhid), lambda i: (0, 0)),
            pl.BlockSpec((nhid, 2 * nhid), lambda i: (0, 0)),
            pl.BlockSpec((bm, n), adj_map),
        ],
        out_specs=[
            pl.BlockSpec((bm, 2 * nhid), muvar_map),
            pl.BlockSpec((bm, n), recon_map),
        ],
        out_shape=[
            jax.ShapeDtypeStruct((n, 2 * nhid), jnp.float32),
            jax.ShapeDtypeStruct((n, n), jnp.float32),
        ],
        scratch_shapes=[
            pltpu.VMEM((n, nhid), jnp.float32),
            pltpu.VMEM((n, 2 * nhid), jnp.float32),
            pltpu.VMEM((n, nhid), jnp.float32),
        ],
        compiler_params=pltpu.CompilerParams(
            dimension_semantics=("arbitrary",)),
    )(x, W1, wc, adj)

    mu = muvar[:, :nhid]
    logvar = muvar[:, nhid:]
    return (recon, mu, logvar)
